# baseline (speedup 1.0000x reference)
import functools

import jax
import jax.numpy as jnp
from jax import lax
from jax.experimental import pallas as pl
from jax.experimental.pallas import tpu as pltpu
from jax.experimental.pallas import tpu_sc as plsc

_N = 10000
_E = 320000
_D = 128
_NP = _N + 8

_NC = 2
_NS = 16
_NW = _NC * _NS
_EPW = _E // _NW
_CHUNK = 128
_NCHUNK = 80
_EPWP = _NCHUNK * _CHUNK
_RPT = 1000
_NEXP = _N // _RPT

_mesh = plsc.VectorSubcoreMesh(core_axis_name="c", subcore_axis_name="s")


@functools.partial(
    pl.kernel,
    out_type=jax.ShapeDtypeStruct((_NW, _N * 8), jnp.float32),
    mesh=_mesh,
    scratch_types=[
        pltpu.VMEM((_EPW,), jnp.int32),
        pltpu.VMEM((_N * 8,), jnp.float32),
    ],
    compiler_params=pltpu.CompilerParams(needs_layout_passes=False),
)
def _deg_kernel(col_hbm, zeros_hbm, out_hbm, col_v, cnt_v):
    cid = lax.axis_index("c")
    sid = lax.axis_index("s")
    wid = cid * _NS + sid
    pltpu.sync_copy(col_hbm.at[wid], col_v)
    pltpu.sync_copy(zeros_hbm, cnt_v)

    lane = lax.broadcasted_iota(jnp.int32, (16,), 0)
    slot_base = (lane & 7) * _N
    lo = lane < 8
    ones = jnp.ones((16,), jnp.float32)

    def body(i, carry):
        c = col_v[pl.ds(i * 16, 16)]
        idx = slot_base + c
        plsc.addupdate_scatter(cnt_v, [idx], ones, mask=lo)
        plsc.addupdate_scatter(cnt_v, [idx], ones, mask=jnp.logical_not(lo))
        return carry

    lax.fori_loop(0, _EPW // 16, body, 0)
    pltpu.sync_copy(cnt_v, out_hbm.at[wid])


@functools.partial(
    pl.kernel,
    out_type=jax.ShapeDtypeStruct((_NC, _N, _D), jnp.float32),
    mesh=_mesh,
    scratch_types=[
        pltpu.VMEM((_NCHUNK, _CHUNK), jnp.int32),
        pltpu.VMEM((_NCHUNK, _CHUNK), jnp.int32),
        pltpu.VMEM((_CHUNK, _D), jnp.float32),
        pltpu.VMEM_SHARED((_NP, _D), jnp.float32),
        pltpu.SemaphoreType.DMA,
    ],
)
def _msg_kernel(g_hbm, row_hbm, col_hbm, zeros_hbm, out_hbm,
                col_v, row_v, buf, acc_sh, gsem):
    cid = lax.axis_index("c")
    sid = lax.axis_index("s")
    wid = cid * _NS + sid
    pltpu.sync_copy(col_hbm.at[wid], col_v)
    pltpu.sync_copy(row_hbm.at[wid], row_v)

    @pl.when(sid < _NEXP)
    def _zero():
        pltpu.sync_copy(zeros_hbm.at[pl.ds(sid * _RPT, _RPT)],
                        acc_sh.at[pl.ds(sid * _RPT, _RPT)])

    plsc.subcore_barrier()

    def body(jj, carry):
        pltpu.async_copy(g_hbm.at[row_v.at[jj]], buf, gsem).wait()
        pltpu.sync_copy(buf, acc_sh.at[col_v.at[jj]], add=True)
        return carry

    lax.fori_loop(0, _NCHUNK, body, 0)
    plsc.subcore_barrier()

    @pl.when(sid < _NEXP)
    def _export():
        pltpu.sync_copy(acc_sh.at[pl.ds(sid * _RPT, _RPT)],
                        out_hbm.at[cid, pl.ds(sid * _RPT, _RPT)])


_ROWS = 2000


def _mm_body(x_ref, w_ref, h_ref):
    h_ref[...] = jnp.dot(x_ref[...], w_ref[...],
                         preferred_element_type=jnp.float32)


def _p_body(parts_ref, p_ref):
    deg = 1.0 + jnp.sum(parts_ref[...], axis=(0, 1))
    p_ref[...] = lax.rsqrt(deg)[:, None]


def _norm_body(p_ref, h_ref, g_ref):
    g_ref[...] = p_ref[...] * h_ref[...]


def _head_body(p_ref, acc_ref, g_ref, b_ref, w2_ref, b2_ref, y_ref):
    t = p_ref[...] * (acc_ref[0] + acc_ref[1] + g_ref[...]) + b_ref[...]
    t = jnp.maximum(t, 0.0)
    y_ref[...] = jnp.dot(t, w2_ref[...],
                         preferred_element_type=jnp.float32) + b2_ref[...]


def kernel(x, edge_index, edge_attr, W, b, W2, b2):
    del edge_attr
    row = edge_index[0].astype(jnp.int32).reshape(_NW, _EPW)
    col = edge_index[1].astype(jnp.int32).reshape(_NW, _EPW)
    pad = ((0, 0), (0, _EPWP - _EPW))
    row_msg = jnp.pad(row, pad, constant_values=0).reshape(
        _NW, _NCHUNK, _CHUNK)
    col_msg = jnp.pad(col, pad, constant_values=_N).reshape(
        _NW, _NCHUNK, _CHUNK)
    zeros8 = jnp.zeros((_N * 8,), jnp.float32)
    zerosd = jnp.zeros((_N, _D), jnp.float32)

    deg_parts = _deg_kernel(col, zeros8).reshape(_NW, 8, _N)
    p = pl.pallas_call(
        _p_body,
        out_shape=jax.ShapeDtypeStruct((_N, 1), jnp.float32),
    )(deg_parts)

    h = pl.pallas_call(
        _mm_body,
        grid=(_N // _ROWS,),
        in_specs=[
            pl.BlockSpec((_ROWS, _D), lambda i: (i, 0)),
            pl.BlockSpec((_D, _D), lambda i: (0, 0)),
        ],
        out_specs=pl.BlockSpec((_ROWS, _D), lambda i: (i, 0)),
        out_shape=jax.ShapeDtypeStruct((_N, _D), jnp.float32),
    )(x, W)

    g = pl.pallas_call(
        _norm_body,
        grid=(_N // _ROWS,),
        in_specs=[
            pl.BlockSpec((_ROWS, 1), lambda i: (i, 0)),
            pl.BlockSpec((_ROWS, _D), lambda i: (i, 0)),
        ],
        out_specs=pl.BlockSpec((_ROWS, _D), lambda i: (i, 0)),
        out_shape=jax.ShapeDtypeStruct((_N, _D), jnp.float32),
    )(p, h)

    acc = _msg_kernel(g, row_msg, col_msg, zerosd)

    y = pl.pallas_call(
        _head_body,
        grid=(_N // _ROWS,),
        in_specs=[
            pl.BlockSpec((_ROWS, 1), lambda i: (i, 0)),
            pl.BlockSpec((_NC, _ROWS, _D), lambda i: (0, i, 0)),
            pl.BlockSpec((_ROWS, _D), lambda i: (i, 0)),
            pl.BlockSpec((_D,), lambda i: (0,)),
            pl.BlockSpec((_D, 1), lambda i: (0, 0)),
            pl.BlockSpec((1,), lambda i: (0,)),
        ],
        out_specs=pl.BlockSpec((_ROWS, 1), lambda i: (i, 0)),
        out_shape=jax.ShapeDtypeStruct((_N, 1), jnp.float32),
    )(p, acc, g, b, W2, b2)
    return y


# speedup vs baseline: 1.8491x; 1.8491x over previous
import functools

import jax
import jax.numpy as jnp
from jax import lax
from jax.experimental import pallas as pl
from jax.experimental.pallas import tpu as pltpu
from jax.experimental.pallas import tpu_sc as plsc

_N = 10000
_E = 320000
_D = 128
_NP = _N + 16

_NC = 2
_NS = 16
_NW = _NC * _NS
_EPW = _E // _NW
_CHUNK = 128
_NCHUNK = 80
_EPWP = _NCHUNK * _CHUNK
_RPT = 1000
_NEXP = _N // _RPT

_mesh = plsc.VectorSubcoreMesh(core_axis_name="c", subcore_axis_name="s")


@functools.partial(
    pl.kernel,
    out_type=jax.ShapeDtypeStruct((_NW, _N * 8), jnp.float32),
    mesh=_mesh,
    scratch_types=[
        pltpu.VMEM((_EPW,), jnp.int32),
        pltpu.VMEM((_N * 8,), jnp.float32),
    ],
    compiler_params=pltpu.CompilerParams(needs_layout_passes=False),
)
def _deg_kernel(col_hbm, zeros_hbm, out_hbm, col_v, cnt_v):
    cid = lax.axis_index("c")
    sid = lax.axis_index("s")
    wid = cid * _NS + sid
    pltpu.sync_copy(col_hbm.at[wid], col_v)
    pltpu.sync_copy(zeros_hbm, cnt_v)

    lane = lax.broadcasted_iota(jnp.int32, (16,), 0)
    slot_base = (lane & 7) * _N
    lo = lane < 8
    ones = jnp.ones((16,), jnp.float32)

    def body(i, carry):
        c = col_v[pl.ds(i * 16, 16)]
        idx = slot_base + c
        plsc.addupdate_scatter(cnt_v, [idx], ones, mask=lo)
        plsc.addupdate_scatter(cnt_v, [idx], ones, mask=jnp.logical_not(lo))
        return carry

    lax.fori_loop(0, _EPW // 16, body, 0)
    pltpu.sync_copy(cnt_v, out_hbm.at[wid])


@functools.partial(
    pl.kernel,
    out_type=jax.ShapeDtypeStruct((_NC, _N, _D), jnp.float32),
    mesh=_mesh,
    scratch_types=[
        pltpu.VMEM((_NCHUNK, _CHUNK), jnp.int32),
        pltpu.VMEM((_NCHUNK, _CHUNK), jnp.int32),
        pltpu.VMEM((_CHUNK, _D), jnp.float32),
        pltpu.VMEM_SHARED((_NP, _D), jnp.float32),
        pltpu.SemaphoreType.DMA,
    ],
)
def _msg_kernel(g_hbm, row_hbm, col_hbm, zeros_hbm, out_hbm,
                col_v, row_v, buf, acc_sh, gsem):
    cid = lax.axis_index("c")
    sid = lax.axis_index("s")
    wid = cid * _NS + sid
    pltpu.sync_copy(col_hbm.at[wid], col_v)
    pltpu.sync_copy(row_hbm.at[wid], row_v)

    @pl.when(sid < _NEXP)
    def _zero():
        pltpu.sync_copy(zeros_hbm.at[pl.ds(sid * _RPT, _RPT)],
                        acc_sh.at[pl.ds(sid * _RPT, _RPT)])

    plsc.subcore_barrier()

    def body(jj, carry):
        pltpu.async_copy(g_hbm.at[row_v.at[jj]], buf, gsem).wait()
        pltpu.sync_copy(buf, acc_sh.at[col_v.at[jj]], add=True)
        return carry

    lax.fori_loop(0, _NCHUNK, body, 0)
    plsc.subcore_barrier()

    @pl.when(sid < _NEXP)
    def _export():
        pltpu.sync_copy(acc_sh.at[pl.ds(sid * _RPT, _RPT)],
                        out_hbm.at[cid, pl.ds(sid * _RPT, _RPT)])


_ROWS = 2000


def _mm_body(x_ref, w_ref, h_ref):
    h_ref[...] = jnp.dot(x_ref[...], w_ref[...],
                         preferred_element_type=jnp.float32)


def _p_body(parts_ref, p_ref):
    deg = 1.0 + jnp.sum(parts_ref[...], axis=(0, 1))
    p_ref[...] = lax.rsqrt(deg)[:, None]


def _norm_body(p_ref, h_ref, g_ref):
    g_ref[...] = p_ref[...] * h_ref[...]


def _head_body(p_ref, acc_ref, g_ref, b_ref, w2_ref, b2_ref, y_ref):
    t = p_ref[...] * (acc_ref[0] + acc_ref[1] + g_ref[...]) + b_ref[...]
    t = jnp.maximum(t, 0.0)
    y_ref[...] = jnp.dot(t, w2_ref[...],
                         preferred_element_type=jnp.float32) + b2_ref[...]


def kernel(x, edge_index, edge_attr, W, b, W2, b2):
    del edge_attr
    row = edge_index[0].astype(jnp.int32).reshape(_NW, _EPW)
    col = edge_index[1].astype(jnp.int32).reshape(_NW, _EPW)
    tile_id = jnp.arange(_NW, dtype=jnp.int32) % _NS
    padblk = jnp.broadcast_to(tile_id[:, None], (_NW, _EPWP - _EPW))
    row_msg = jnp.concatenate([row, padblk], axis=1).reshape(
        _NW, _NCHUNK, _CHUNK)
    col_msg = jnp.concatenate([col, _N + padblk], axis=1).reshape(
        _NW, _NCHUNK, _CHUNK)
    zeros8 = jnp.zeros((_N * 8,), jnp.float32)
    zerosd = jnp.zeros((_N, _D), jnp.float32)

    deg_parts = _deg_kernel(col, zeros8).reshape(_NW, 8, _N)
    p = pl.pallas_call(
        _p_body,
        out_shape=jax.ShapeDtypeStruct((_N, 1), jnp.float32),
    )(deg_parts)

    h = pl.pallas_call(
        _mm_body,
        grid=(_N // _ROWS,),
        in_specs=[
            pl.BlockSpec((_ROWS, _D), lambda i: (i, 0)),
            pl.BlockSpec((_D, _D), lambda i: (0, 0)),
        ],
        out_specs=pl.BlockSpec((_ROWS, _D), lambda i: (i, 0)),
        out_shape=jax.ShapeDtypeStruct((_N, _D), jnp.float32),
    )(x, W)

    g = pl.pallas_call(
        _norm_body,
        grid=(_N // _ROWS,),
        in_specs=[
            pl.BlockSpec((_ROWS, 1), lambda i: (i, 0)),
            pl.BlockSpec((_ROWS, _D), lambda i: (i, 0)),
        ],
        out_specs=pl.BlockSpec((_ROWS, _D), lambda i: (i, 0)),
        out_shape=jax.ShapeDtypeStruct((_N, _D), jnp.float32),
    )(p, h)

    acc = _msg_kernel(g, row_msg, col_msg, zerosd)

    y = pl.pallas_call(
        _head_body,
        grid=(_N // _ROWS,),
        in_specs=[
            pl.BlockSpec((_ROWS, 1), lambda i: (i, 0)),
            pl.BlockSpec((_NC, _ROWS, _D), lambda i: (0, i, 0)),
            pl.BlockSpec((_ROWS, _D), lambda i: (i, 0)),
            pl.BlockSpec((_D,), lambda i: (0,)),
            pl.BlockSpec((_D, 1), lambda i: (0, 0)),
            pl.BlockSpec((1,), lambda i: (0,)),
        ],
        out_specs=pl.BlockSpec((_ROWS, 1), lambda i: (i, 0)),
        out_shape=jax.ShapeDtypeStruct((_N, 1), jnp.float32),
    )(p, acc, g, b, W2, b2)
    return y


# speedup vs baseline: 1.8753x; 1.0142x over previous
import functools

import jax
import jax.numpy as jnp
from jax import lax
from jax.experimental import pallas as pl
from jax.experimental.pallas import tpu as pltpu
from jax.experimental.pallas import tpu_sc as plsc

_N = 10000
_E = 320000
_D = 128
_NP = _N + 16

_NC = 2
_NS = 16
_NW = _NC * _NS
_EPW = _E // _NW
_CHUNK = 128
_NCHUNK = 80
_EPWP = _NCHUNK * _CHUNK
_RPT = 1000
_NEXP = _N // _RPT

_mesh = plsc.VectorSubcoreMesh(core_axis_name="c", subcore_axis_name="s")


@functools.partial(
    pl.kernel,
    out_type=jax.ShapeDtypeStruct((_NW, _N * 8), jnp.float32),
    mesh=_mesh,
    scratch_types=[
        pltpu.VMEM((_EPW,), jnp.int32),
        pltpu.VMEM((_N * 8,), jnp.float32),
    ],
    compiler_params=pltpu.CompilerParams(needs_layout_passes=False),
)
def _deg_kernel(col_hbm, zeros_hbm, out_hbm, col_v, cnt_v):
    cid = lax.axis_index("c")
    sid = lax.axis_index("s")
    wid = cid * _NS + sid
    pltpu.sync_copy(col_hbm.at[wid], col_v)
    pltpu.sync_copy(zeros_hbm, cnt_v)

    lane = lax.broadcasted_iota(jnp.int32, (16,), 0)
    slot_base = (lane & 7) * _N
    lo = lane < 8
    ones = jnp.ones((16,), jnp.float32)

    def body(i, carry):
        c = col_v[pl.ds(i * 16, 16)]
        idx = slot_base + c
        plsc.addupdate_scatter(cnt_v, [idx], ones, mask=lo)
        plsc.addupdate_scatter(cnt_v, [idx], ones, mask=jnp.logical_not(lo))
        return carry

    lax.fori_loop(0, _EPW // 16, body, 0)
    pltpu.sync_copy(cnt_v, out_hbm.at[wid])


@functools.partial(
    pl.kernel,
    out_type=jax.ShapeDtypeStruct((_NC, _N, _D), jnp.float32),
    mesh=_mesh,
    scratch_types=[
        pltpu.VMEM((_NCHUNK, _CHUNK), jnp.int32),
        pltpu.VMEM((_NCHUNK, _CHUNK), jnp.int32),
        pltpu.VMEM((_CHUNK, _D), jnp.float32),
        pltpu.VMEM_SHARED((_NP, _D), jnp.float32),
        pltpu.SemaphoreType.DMA,
    ],
)
def _msg_kernel(g_hbm, row_hbm, col_hbm, zeros_hbm, out_hbm,
                col_v, row_v, buf, acc_sh, gsem):
    cid = lax.axis_index("c")
    sid = lax.axis_index("s")
    wid = cid * _NS + sid
    pltpu.sync_copy(col_hbm.at[wid], col_v)
    pltpu.sync_copy(row_hbm.at[wid], row_v)

    @pl.when(sid < _NEXP)
    def _zero():
        pltpu.sync_copy(zeros_hbm.at[pl.ds(sid * _RPT, _RPT)],
                        acc_sh.at[pl.ds(sid * _RPT, _RPT)])

    plsc.subcore_barrier()

    def body(jj, carry):
        pltpu.async_copy(g_hbm.at[row_v.at[jj]], buf, gsem).wait()
        pltpu.sync_copy(buf, acc_sh.at[col_v.at[jj]], add=True)
        return carry

    lax.fori_loop(0, _NCHUNK, body, 0)
    plsc.subcore_barrier()

    @pl.when(sid < _NEXP)
    def _export():
        pltpu.sync_copy(acc_sh.at[pl.ds(sid * _RPT, _RPT)],
                        out_hbm.at[cid, pl.ds(sid * _RPT, _RPT)])


_ROWS = 2000


def _p_body(parts_ref, p_ref):
    deg = 1.0 + jnp.sum(parts_ref[...], axis=(0, 1))
    p_ref[...] = lax.rsqrt(deg)[:, None]


def _gmm_body(p_ref, x_ref, w_ref, g_ref):
    g_ref[...] = p_ref[...] * jnp.dot(x_ref[...], w_ref[...],
                                      preferred_element_type=jnp.float32)


def _head_body(p_ref, acc_ref, g_ref, b_ref, w2_ref, b2_ref, y_ref):
    t = p_ref[...] * (acc_ref[0] + acc_ref[1] + g_ref[...]) + b_ref[...]
    t = jnp.maximum(t, 0.0)
    y_ref[...] = jnp.dot(t, w2_ref[...],
                         preferred_element_type=jnp.float32) + b2_ref[...]


def kernel(x, edge_index, edge_attr, W, b, W2, b2):
    del edge_attr
    row = edge_index[0].astype(jnp.int32).reshape(_NW, _EPW)
    col = edge_index[1].astype(jnp.int32).reshape(_NW, _EPW)
    tile_id = jnp.arange(_NW, dtype=jnp.int32) % _NS
    padblk = jnp.broadcast_to(tile_id[:, None], (_NW, _EPWP - _EPW))
    row_msg = jnp.concatenate([row, padblk], axis=1).reshape(
        _NW, _NCHUNK, _CHUNK)
    col_msg = jnp.concatenate([col, _N + padblk], axis=1).reshape(
        _NW, _NCHUNK, _CHUNK)
    zeros8 = jnp.zeros((_N * 8,), jnp.float32)
    zerosd = jnp.zeros((_N, _D), jnp.float32)

    deg_parts = _deg_kernel(col, zeros8).reshape(_NW, 8, _N)
    p = pl.pallas_call(
        _p_body,
        out_shape=jax.ShapeDtypeStruct((_N, 1), jnp.float32),
    )(deg_parts)

    g = pl.pallas_call(
        _gmm_body,
        grid=(_N // _ROWS,),
        in_specs=[
            pl.BlockSpec((_ROWS, 1), lambda i: (i, 0)),
            pl.BlockSpec((_ROWS, _D), lambda i: (i, 0)),
            pl.BlockSpec((_D, _D), lambda i: (0, 0)),
        ],
        out_specs=pl.BlockSpec((_ROWS, _D), lambda i: (i, 0)),
        out_shape=jax.ShapeDtypeStruct((_N, _D), jnp.float32),
    )(p, x, W)

    acc = _msg_kernel(g, row_msg, col_msg, zerosd)

    y = pl.pallas_call(
        _head_body,
        grid=(_N // _ROWS,),
        in_specs=[
            pl.BlockSpec((_ROWS, 1), lambda i: (i, 0)),
            pl.BlockSpec((_NC, _ROWS, _D), lambda i: (0, i, 0)),
            pl.BlockSpec((_ROWS, _D), lambda i: (i, 0)),
            pl.BlockSpec((_D,), lambda i: (0,)),
            pl.BlockSpec((_D, 1), lambda i: (0, 0)),
            pl.BlockSpec((1,), lambda i: (0,)),
        ],
        out_specs=pl.BlockSpec((_ROWS, 1), lambda i: (i, 0)),
        out_shape=jax.ShapeDtypeStruct((_N, 1), jnp.float32),
    )(p, acc, g, b, W2, b2)
    return y
